# baseline (speedup 1.0000x reference)
import functools

import jax
import jax.numpy as jnp
from jax import lax
from jax.experimental import pallas as pl
from jax.experimental.pallas import tpu as pltpu
from jax.experimental.pallas import tpu_sc as plsc

_B = 16384
_D = 32
_L = 16
_NC = 2
_NS = 16
_NW = _NC * _NS
_BPW = _B // _NW
_CI = 64
_NCH = _BPW // _CI
_ROWS = _CI * _D
_LPD = 1000000 // 16

_mesh = plsc.VectorSubcoreMesh(core_axis_name="c", subcore_axis_name="s")


def _body(ub_hbm, ib_hbm, ut_hbm, it_hbm, out_hbm,
          uidx_v, iidx_v, us_v, is_v, ubuf, ibuf, out_v, sem_u, sem_i):
    wid = lax.axis_index("s") * _NC + lax.axis_index("c")
    base = wid * _BPW

    pltpu.sync_copy(ub_hbm.at[pl.ds(base, _BPW)], uidx_v)
    pltpu.sync_copy(ib_hbm.at[pl.ds(base, _BPW)], iidx_v)

    lanes = lax.iota(jnp.int32, _L)

    def chunk_body(ch, carry):
        cb = ch * _CI

        def mk_rows(v, carry2):
            qu = uidx_v[pl.ds(cb + v * _L, _L)] >> 4
            qi = iidx_v[pl.ds(cb + v * _L, _L)] >> 4
            for c in range(_D):
                us_v[pl.ds(c * _CI + v * _L, _L)] = qu + c * _LPD
                is_v[pl.ds(c * _CI + v * _L, _L)] = qi + c * _LPD
            return carry2

        lax.fori_loop(0, _CI // _L, mk_rows, 0)

        cu = pltpu.async_copy(ut_hbm.at[us_v], ubuf, sem_u)
        ci = pltpu.async_copy(it_hbm.at[is_v], ibuf, sem_i)
        cu.wait()
        ci.wait()

        def group(g, carry2):
            b = cb + g * _L
            ru = uidx_v[pl.ds(b, _L)]
            ri = iidx_v[pl.ds(b, _L)]
            rows = g * _L + lanes
            wu = ru & 15
            wi = ri & 15
            acc = jnp.zeros((_L,), jnp.float32)
            for c in range(_D):
                uu = plsc.load_gather(ubuf, [rows + c * _CI, wu])
                vv = plsc.load_gather(ibuf, [rows + c * _CI, wi])
                acc = acc + uu * vv
            out_v[pl.ds(b, _L)] = acc
            return carry2

        lax.fori_loop(0, _CI // _L, group, 0)
        return carry

    lax.fori_loop(0, _NCH, chunk_body, 0)

    pltpu.sync_copy(out_v, out_hbm.at[pl.ds(base, _BPW)])


@jax.jit
def _run(user_batch, item_batch, ut16, it16):
    k = functools.partial(
        pl.kernel,
        out_type=jax.ShapeDtypeStruct((_B,), jnp.float32),
        mesh=_mesh,
        scratch_types=[
            pltpu.VMEM((_BPW,), jnp.int32),
            pltpu.VMEM((_BPW,), jnp.int32),
            pltpu.VMEM((_ROWS,), jnp.int32),
            pltpu.VMEM((_ROWS,), jnp.int32),
            pltpu.VMEM((_ROWS, _L), jnp.float32),
            pltpu.VMEM((_ROWS, _L), jnp.float32),
            pltpu.VMEM((_BPW,), jnp.float32),
            pltpu.SemaphoreType.DMA,
            pltpu.SemaphoreType.DMA,
        ],
        compiler_params=pltpu.CompilerParams(
            needs_layout_passes=False, use_tc_tiling_on_sc=False),
    )(_body)
    return k(user_batch, item_batch, ut16, it16)


def kernel(user_batch, item_batch, user_table, item_table):
    return _run(user_batch.astype(jnp.int32), item_batch.astype(jnp.int32),
                user_table.T.reshape(2000000, 16),
                item_table.T.reshape(2000000, 16))
